# baseline (speedup 1.0000x reference)
import functools

import jax
import jax.numpy as jnp
from jax import lax
from jax.experimental import pallas as pl
from jax.experimental.pallas import tpu as pltpu
from jax.experimental.pallas import tpu_sc as plsc

N = 10000
E = 160000
D = 256
DH = 128
NPAD = 10240
NSUB = 16
ESUB = E // NSUB
C = 80
NCH = ESUB // C
RSTRIPE = NPAD // NSUB
RCH = RSTRIPE // C


def _zero16():
    return jnp.zeros((16,), jnp.float32)


def _lane_bcast(v16, i):
    idx = (jnp.zeros((16,), jnp.int32) + i)[:, None]
    dnums = lax.GatherDimensionNumbers(
        offset_dims=(), collapsed_slice_dims=(0,), start_index_map=(0,))
    return lax.gather(v16, idx, dnums, (1,),
                      mode=lax.GatherScatterMode.PROMISE_IN_BOUNDS)


def _make_pass(first: bool):
    mesh = plsc.VectorSubcoreMesh(core_axis_name="c", subcore_axis_name="s")
    n_out = 1 if first else 2
    out_type = ([jax.ShapeDtypeStruct((NPAD, DH), jnp.float32),
                 jax.ShapeDtypeStruct((NPAD, DH), jnp.float32)] +
                [jax.ShapeDtypeStruct((NPAD,), jnp.float32)] * n_out)
    scratch = [
        pltpu.VMEM((ESUB,), jnp.int32),
        pltpu.VMEM((ESUB,), jnp.float32),
        pltpu.VMEM((C, DH), jnp.float32),
        pltpu.VMEM((C, DH), jnp.float32),
        pltpu.VMEM((C,), jnp.int32),
        pltpu.VMEM((C,), jnp.int32),
        pltpu.VMEM((C,), jnp.float32),
        pltpu.VMEM((C,), jnp.float32),
        pltpu.VMEM((C,), jnp.int32),
        pltpu.VMEM((C,), jnp.int32),
        pltpu.VMEM((C,), jnp.float32),
        pltpu.VMEM((C,), jnp.float32),
        pltpu.VMEM((C,), jnp.float32),
        pltpu.VMEM((C,), jnp.float32),
        pltpu.VMEM((C,), jnp.float32),
        pltpu.VMEM((C,), jnp.float32),
        pltpu.VMEM((RSTRIPE,), jnp.float32),
        pltpu.VMEM((16,), jnp.float32),
        pltpu.VMEM_SHARED((NPAD, DH), jnp.float32),
        pltpu.VMEM_SHARED((NPAD,), jnp.float32),
        pltpu.VMEM_SHARED((NPAD,), jnp.float32),
        pltpu.SemaphoreType.DMA,
        pltpu.SemaphoreType.DMA,
        pltpu.SemaphoreType.DMA,
        pltpu.SemaphoreType.DMA,
        pltpu.SemaphoreType.DMA,
    ]

    def body(cid, sid, rc_hbm, vals_hbm, ordp_hbm, np_hbm,
             params_hbm, x0_hbm, x1_hbm, e0_out, e1_out, ord_out, n_out_hbm,
             rc_s, vals_s, xg, xg_b, rows_ch, cols_ch, vch, nvch,
             rows_chb, cols_chb, vch_b, nvch_b, ng_a, ng_b,
             ordch, npch, z1d, params_v,
             acc_sh, ord_sh, nacc_sh, gsem_a, gsem_b, nsem_a, nsem_b, ssem):
        ebase = sid * ESUB
        rbase = sid * RSTRIPE

        def per_core(fn0, fn1):
            @pl.when(cid == 0)
            def _():
                fn0()

            @pl.when(cid == 1)
            def _():
                fn1()

        hpre = [pltpu.async_copy(rc_hbm.at[pl.ds(ebase, ESUB)], rc_s,
                                 gsem_a),
                pltpu.async_copy(vals_hbm.at[pl.ds(ebase, ESUB)], vals_s,
                                 gsem_b)]
        if not first:
            pltpu.sync_copy(params_hbm, params_v)

        def zrow(r, _):
            for dd in range(DH // 16):
                xg[r, pl.ds(16 * dd, 16)] = _zero16()
            return 0
        lax.fori_loop(0, C, zrow, 0)

        def z1(t, _):
            z1d[pl.ds(16 * t, 16)] = _zero16()
            return 0
        lax.fori_loop(0, RSTRIPE // 16, z1, 0)

        hz = [pltpu.async_copy(xg, acc_sh.at[pl.ds(rbase + k * C, C), :],
                               ssem) for k in range(RCH)]
        hz.append(pltpu.async_copy(z1d, ord_sh.at[pl.ds(rbase, RSTRIPE)],
                                   ssem))
        if not first:
            hz.append(pltpu.async_copy(z1d,
                                       nacc_sh.at[pl.ds(rbase, RSTRIPE)],
                                       ssem))
        for h in hpre + hz:
            h.wait()
        plsc.subcore_barrier()

        bufs = ((xg, rows_ch, cols_ch, vch, nvch, ng_a, gsem_a, nsem_a),
                (xg_b, rows_chb, cols_chb, vch_b, nvch_b, ng_b, gsem_b,
                 nsem_b))

        def prep_and_fire(eb, B):
            bxg, brows, bcols, bvch, bnv, bng, gsem, nsem = B
            for t in range(C // 16):
                rc16 = rc_s[pl.ds(eb + 16 * t, 16)]
                brows[pl.ds(16 * t, 16)] = lax.shift_right_logical(rc16, 14)
                bcols[pl.ds(16 * t, 16)] = lax.bitwise_and(rc16, 16383)
                bvch[pl.ds(16 * t, 16)] = vals_s[pl.ds(eb + 16 * t, 16)]
            per_core(lambda: pltpu.async_copy(x0_hbm.at[bcols], bxg, gsem),
                     lambda: pltpu.async_copy(x1_hbm.at[bcols], bxg, gsem))
            if not first:
                pltpu.async_copy(np_hbm.at[bcols], bng, nsem)

        def wait_and_scale(B):
            bxg, brows, bcols, bvch, bnv, bng, gsem, nsem = B
            pltpu.make_async_copy(x0_hbm.at[bcols], bxg, gsem).wait()
            if not first:
                pltpu.make_async_copy(np_hbm.at[bcols], bng, nsem).wait()
                for t in range(C // 16):
                    sl = pl.ds(16 * t, 16)
                    bnv[sl] = bvch[sl] * bng[sl]
            for t in range(C // 16):
                v16g = bvch[pl.ds(16 * t, 16)]

                def scale(e2, _, v16g=v16g, t=t):
                    for u in range(2):
                        e = 2 * e2 + u
                        vb = _lane_bcast(v16g, e)
                        row = 16 * t + e
                        for dd in range(DH // 16):
                            sl = pl.ds(16 * dd, 16)
                            bxg[row, sl] = bxg[row, sl] * vb
                    return 0
                lax.fori_loop(0, 8, scale, 0)

        def scatter_async(B):
            bxg, brows, bcols, bvch, bnv, bng, gsem, nsem = B
            hs = [pltpu.async_copy(bxg, acc_sh.at[brows], ssem, add=True),
                  pltpu.async_copy(bvch, ord_sh.at[brows], ssem, add=True)]
            if not first:
                hs.append(pltpu.async_copy(bnv, nacc_sh.at[brows], ssem,
                                           add=True))
            return hs

        def scatter_sync(B):
            bxg, brows, bcols, bvch, bnv, bng, gsem, nsem = B
            pltpu.sync_copy(bxg, acc_sh.at[brows], add=True)
            pltpu.sync_copy(bvch, ord_sh.at[brows], add=True)
            if not first:
                pltpu.sync_copy(bnv, nacc_sh.at[brows], add=True)

        def wait_scatter(B):
            bxg, brows, bcols, bvch, bnv, bng, gsem, nsem = B
            pltpu.make_async_copy(bxg, acc_sh.at[brows], ssem).wait()
            pltpu.make_async_copy(bvch, ord_sh.at[brows], ssem).wait()
            if not first:
                pltpu.make_async_copy(bnv, nacc_sh.at[brows], ssem).wait()

        def pairchunk(j2, _):
            ea = (2 * j2) * C
            prep_and_fire(ea, bufs[0])

            @pl.when(j2 > 0)
            def _():
                wait_scatter(bufs[1])
            prep_and_fire(ea + C, bufs[1])
            wait_and_scale(bufs[0])
            hs = scatter_async(bufs[0])
            wait_and_scale(bufs[1])
            for h in hs:
                h.wait()
            scatter_async(bufs[1])
            return 0
        lax.fori_loop(0, NCH // 2, pairchunk, 0)
        wait_scatter(bufs[1])
        prep_and_fire((NCH - 1) * C, bufs[0])
        wait_and_scale(bufs[0])
        scatter_sync(bufs[0])
        plsc.subcore_barrier()

        if not first:
            gb = _lane_bcast(params_v[pl.ds(0, 16)], 0)

        def rowchunk(k, _):
            r0 = rbase + k * C
            pltpu.sync_copy(acc_sh.at[pl.ds(r0, C), :], xg)
            per_core(
                lambda: pltpu.sync_copy(x0_hbm.at[pl.ds(r0, C), :], xg_b),
                lambda: pltpu.sync_copy(x1_hbm.at[pl.ds(r0, C), :], xg_b))
            if not first:
                pltpu.sync_copy(ordp_hbm.at[pl.ds(r0, C)], ordch)

            if first:
                def comb(r, _):
                    for dd in range(DH // 16):
                        sl = pl.ds(16 * dd, 16)
                        xg[r, sl] = xg[r, sl] - xg_b[r, sl]
                    return 0
                lax.fori_loop(0, C, comb, 0)
            else:
                for t in range(C // 16):
                    o16 = ordch[pl.ds(16 * t, 16)] + 1.0

                    def comb(r, _, t=t, o16=o16):
                        coef = _lane_bcast(o16, r)
                        row = 16 * t + r
                        for dd in range(DH // 16):
                            sl = pl.ds(16 * dd, 16)
                            xg[row, sl] = gb * (xg[row, sl] -
                                                coef * xg_b[row, sl])
                        return 0
                    lax.fori_loop(0, 16, comb, 0)
            per_core(
                lambda: pltpu.sync_copy(xg, e0_out.at[pl.ds(r0, C), :]),
                lambda: pltpu.sync_copy(xg, e1_out.at[pl.ds(r0, C), :]))

            @pl.when(cid == 0)
            def _():
                pltpu.sync_copy(ord_sh.at[pl.ds(r0, C)], vch)
                pltpu.sync_copy(vch, ord_out.at[pl.ds(r0, C)])
                if not first:
                    pltpu.sync_copy(nacc_sh.at[pl.ds(r0, C)], nvch)
                    pltpu.sync_copy(np_hbm.at[pl.ds(r0, C)], npch)
                    for t in range(C // 16):
                        sl = pl.ds(16 * t, 16)
                        nvch[sl] = gb * (nvch[sl] - npch[sl] - ordch[sl])
                    pltpu.sync_copy(nvch, n_out_hbm.at[pl.ds(r0, C)])
            return 0
        lax.fori_loop(0, RCH, rowchunk, 0)

    @functools.partial(pl.kernel, mesh=mesh, out_type=out_type,
                       scratch_types=scratch)
    def pass_kernel(rc_hbm, vals_hbm, ordp_hbm, np_hbm,
                    params_hbm, x0_hbm, x1_hbm, e0_out, e1_out, ord_out,
                    *rest):
        if first:
            n_out_hbm, scratch_refs = None, rest
        else:
            n_out_hbm, scratch_refs = rest[0], rest[1:]
        (rc_s, vals_s, xg, xg_b, rows_ch, cols_ch, vch, nvch,
         rows_chb, cols_chb, vch_b, nvch_b, ng_a, ng_b, ordch, npch,
         z1d, params_v, acc_sh, ord_sh, nacc_sh,
         gsem_a, gsem_b, nsem_a, nsem_b, ssem) = scratch_refs
        cid = lax.axis_index("c")
        sid = lax.axis_index("s")
        body(cid, sid, rc_hbm, vals_hbm, ordp_hbm, np_hbm, params_hbm,
             x0_hbm, x1_hbm, e0_out, e1_out, ord_out, n_out_hbm,
             rc_s, vals_s, xg, xg_b, rows_ch, cols_ch, vch, nvch,
             rows_chb, cols_chb, vch_b, nvch_b, ng_a, ng_b, ordch, npch,
             z1d, params_v, acc_sh, ord_sh, nacc_sh,
             gsem_a, gsem_b, nsem_a, nsem_b, ssem)

    return pass_kernel


_pass_first = _make_pass(True)
_pass_rest = _make_pass(False)


_TB = 256
_GRID = NPAD // _TB
_K = 100


def _score_body(e0l, e0h, e1l, e1h, e2l, e2h, e3l, e3h, emb, n0, n1, n2, n3,
                u, nmc, scores_out, cand_out, sacc):
    step = pl.program_id(0)
    lo = e0l[...] + e1l[...] + e2l[...] + e3l[...]
    hi = e0h[...] + e1h[...] + e2h[...] + e3h[...]
    nsum = n0[...] + n1[...] + n2[...] + n3[...] + 1e-8
    lo = lo / nsum
    hi = hi / nsum
    el = emb[:, :DH]
    eh = emb[:, DH:]
    dot = jnp.sum(lo * el + hi * eh, axis=1)
    ns = jnp.sqrt(jnp.sum(lo * lo + hi * hi, axis=1))
    ne = jnp.sqrt(jnp.sum(emb[...] * emb[...], axis=1))
    g = -jnp.log(-jnp.log(u[...][:, 0]))
    row = step * _TB + lax.broadcasted_iota(jnp.int32, (_TB,), 0)
    s = dot / (jnp.maximum(ns, 1e-12) * jnp.maximum(ne, 1e-12)) + g
    s = jnp.where(row < N, s, -1e30)
    scores_out[...] = s[:, None]
    sacc[step] = s

    @pl.when(step == _GRID - 1)
    def _():
        flat_iota = (lax.broadcasted_iota(jnp.int32, (_GRID, _TB), 0) * _TB +
                     lax.broadcasted_iota(jnp.int32, (_GRID, _TB), 1))
        lane = lax.broadcasted_iota(jnp.int32, (8, 128), 1)
        rowz = lax.broadcasted_iota(jnp.int32, (8, 128), 0)

        def pick(k, carry):
            sv, cand = carry
            m = jnp.max(sv)
            idx = jnp.min(jnp.where(sv == m, flat_iota, jnp.int32(2 ** 30)))
            cand = jnp.where((rowz == 0) & (lane == k), idx, cand)
            sv = jnp.where(flat_iota == idx, -3e30, sv)
            return sv, cand

        _, cand = lax.fori_loop(0, _K, pick,
                                (sacc[...], jnp.full((8, 128), -1,
                                                     jnp.int32)))
        cand = jnp.where(lane < nmc[0, 0], cand, -1)
        cand_out[...] = cand[0:1, :]


def _score_call(parts, embp, nvecs, u, nmc):
    row_spec = pl.BlockSpec((_TB, DH), lambda i: (i, 0))
    full_spec = pl.BlockSpec((_TB, D), lambda i: (i, 0))
    col_spec = pl.BlockSpec((_TB, 1), lambda i: (i, 0))
    one_spec = pl.BlockSpec((1, 1), lambda i: (0, 0))
    return pl.pallas_call(
        _score_body,
        grid=(_GRID,),
        in_specs=[row_spec] * 8 + [full_spec] + [col_spec] * 5 + [one_spec],
        out_specs=[pl.BlockSpec((_TB, 1), lambda i: (i, 0)),
                   pl.BlockSpec((1, 128), lambda i: (0, 0))],
        out_shape=[jax.ShapeDtypeStruct((NPAD, 1), jnp.float32),
                   jax.ShapeDtypeStruct((1, 128), jnp.int32)],
        scratch_shapes=[pltpu.VMEM((_GRID, _TB), jnp.float32)],
    )(*parts, embp, *nvecs, u, nmc)


def kernel(edge_index, adj_values, embeds, mask_depth, num_mask_cand):
    rows = edge_index[0].astype(jnp.int32)
    cols = edge_index[1].astype(jnp.int32)
    adj = adj_values.astype(jnp.float32)

    embp = jnp.pad(embeds, ((0, NPAD - N), (0, 0)))
    x0 = embp[:, :DH]
    x1 = embp[:, DH:]

    fs = []
    for i in range(3):
        keep = 0.5 ** (i + 1)
        un = jax.random.uniform(jax.random.fold_in(jax.random.key(1), i),
                                (E,))
        active = i < mask_depth
        f = jnp.where(active, jnp.where(un < keep, 1.0 / keep, 0.0), 1.0)
        fs.append(f)
    fc = [fs[0], fs[0] * fs[1], fs[0] * fs[1] * fs[2]]
    gflags = [jnp.where(i < mask_depth, 1.0, 0.0) for i in range(3)]
    params = [jnp.zeros((16,), jnp.float32).at[0].set(g) for g in gflags]
    unoise = jax.random.uniform(jax.random.key(2), (N,))
    up = jnp.pad(unoise, (0, NPAD - N), constant_values=0.5)[:, None]

    dummy_f = jnp.zeros((16,), jnp.float32)
    dummy_n = jnp.zeros((NPAD,), jnp.float32)
    rc = rows * 16384 + cols

    e0l, e0h, ord0 = _pass_first(rc, adj, dummy_n, dummy_n,
                                 dummy_f, x0, x1)
    n0 = ord0
    e1l, e1h, ord1, n1 = _pass_rest(rc, adj * fc[0], ord0, n0,
                                    params[0], e0l, e0h)
    e2l, e2h, ord2, n2 = _pass_rest(rc, adj * fc[1], ord1, n1,
                                    params[1], e1l, e1h)
    e3l, e3h, ord3, n3 = _pass_rest(rc, adj * fc[2], ord2, n2,
                                    params[2], e2l, e2h)

    nmc = jnp.full((1, 1), num_mask_cand, jnp.int32)
    scores_p, cand = _score_call(
        [e0l, e0h, e1l, e1h, e2l, e2h, e3l, e3h], embp,
        [n0[:, None], n1[:, None], n2[:, None], n3[:, None]], up, nmc)
    return scores_p[:N, 0], cand[0, :_K]


# speedup vs baseline: 1.0048x; 1.0048x over previous
import functools

import jax
import jax.numpy as jnp
from jax import lax
from jax.experimental import pallas as pl
from jax.experimental.pallas import tpu as pltpu
from jax.experimental.pallas import tpu_sc as plsc

N = 10000
E = 160000
D = 256
DH = 128
NPAD = 10240
NSUB = 16
ESUB = E // NSUB
C = 80
NCH = ESUB // C
RSTRIPE = NPAD // NSUB
RCH = RSTRIPE // C


def _zero16():
    return jnp.zeros((16,), jnp.float32)


def _lane_bcast(v16, i):
    idx = (jnp.zeros((16,), jnp.int32) + i)[:, None]
    dnums = lax.GatherDimensionNumbers(
        offset_dims=(), collapsed_slice_dims=(0,), start_index_map=(0,))
    return lax.gather(v16, idx, dnums, (1,),
                      mode=lax.GatherScatterMode.PROMISE_IN_BOUNDS)


def _make_pass(first: bool):
    mesh = plsc.VectorSubcoreMesh(core_axis_name="c", subcore_axis_name="s")
    n_out = 1 if first else 2
    out_type = ([jax.ShapeDtypeStruct((NPAD, DH), jnp.float32),
                 jax.ShapeDtypeStruct((NPAD, DH), jnp.float32)] +
                [jax.ShapeDtypeStruct((NPAD,), jnp.float32)] * n_out)
    scratch = [
        pltpu.VMEM((ESUB,), jnp.int32),
        pltpu.VMEM((ESUB,), jnp.float32),
        pltpu.VMEM((C, DH), jnp.float32),
        pltpu.VMEM((C, DH), jnp.float32),
        pltpu.VMEM((C,), jnp.int32),
        pltpu.VMEM((C,), jnp.int32),
        pltpu.VMEM((C,), jnp.float32),
        pltpu.VMEM((C,), jnp.float32),
        pltpu.VMEM((C,), jnp.int32),
        pltpu.VMEM((C,), jnp.int32),
        pltpu.VMEM((C,), jnp.float32),
        pltpu.VMEM((C,), jnp.float32),
        pltpu.VMEM((C,), jnp.float32),
        pltpu.VMEM((C,), jnp.float32),
        pltpu.VMEM((C,), jnp.float32),
        pltpu.VMEM((C,), jnp.float32),
        pltpu.VMEM((RSTRIPE,), jnp.float32),
        pltpu.VMEM((16,), jnp.float32),
        pltpu.VMEM_SHARED((NPAD, DH), jnp.float32),
        pltpu.VMEM_SHARED((NPAD,), jnp.float32),
        pltpu.VMEM_SHARED((NPAD,), jnp.float32),
        pltpu.SemaphoreType.DMA,
        pltpu.SemaphoreType.DMA,
        pltpu.SemaphoreType.DMA,
        pltpu.SemaphoreType.DMA,
        pltpu.SemaphoreType.DMA,
    ]

    def body(cid, sid, rc_hbm, vals_hbm, ordp_hbm, np_hbm,
             params_hbm, x0_hbm, x1_hbm, e0_out, e1_out, ord_out, n_out_hbm,
             rc_s, vals_s, xg, xg_b, rows_ch, cols_ch, vch, nvch,
             rows_chb, cols_chb, vch_b, nvch_b, ng_a, ng_b,
             ordch, npch, z1d, params_v,
             acc_sh, ord_sh, nacc_sh, gsem_a, gsem_b, nsem_a, nsem_b, ssem):
        ebase = sid * ESUB
        rbase = sid * RSTRIPE

        def per_core(fn0, fn1):
            @pl.when(cid == 0)
            def _():
                fn0()

            @pl.when(cid == 1)
            def _():
                fn1()

        hpre = [pltpu.async_copy(rc_hbm.at[pl.ds(ebase, ESUB)], rc_s,
                                 gsem_a),
                pltpu.async_copy(vals_hbm.at[pl.ds(ebase, ESUB)], vals_s,
                                 gsem_b)]
        if not first:
            pltpu.sync_copy(params_hbm, params_v)

        def zrow(r, _):
            for dd in range(DH // 16):
                xg[r, pl.ds(16 * dd, 16)] = _zero16()
            return 0
        lax.fori_loop(0, C, zrow, 0)

        def z1(t, _):
            z1d[pl.ds(16 * t, 16)] = _zero16()
            return 0
        lax.fori_loop(0, RSTRIPE // 16, z1, 0)

        hz = [pltpu.async_copy(xg, acc_sh.at[pl.ds(rbase + k * C, C), :],
                               ssem) for k in range(RCH)]
        hz.append(pltpu.async_copy(z1d, ord_sh.at[pl.ds(rbase, RSTRIPE)],
                                   ssem))
        if not first:
            hz.append(pltpu.async_copy(z1d,
                                       nacc_sh.at[pl.ds(rbase, RSTRIPE)],
                                       ssem))
        for h in hpre + hz:
            h.wait()
        plsc.subcore_barrier()

        bufs = ((xg, rows_ch, cols_ch, vch, nvch, ng_a, gsem_a, nsem_a),
                (xg_b, rows_chb, cols_chb, vch_b, nvch_b, ng_b, gsem_b,
                 nsem_b))

        def prep_and_fire(eb, B):
            bxg, brows, bcols, bvch, bnv, bng, gsem, nsem = B
            for t in range(C // 16):
                rc16 = rc_s[pl.ds(eb + 16 * t, 16)]
                brows[pl.ds(16 * t, 16)] = lax.shift_right_logical(rc16, 14)
                bcols[pl.ds(16 * t, 16)] = lax.bitwise_and(rc16, 16383)
                bvch[pl.ds(16 * t, 16)] = vals_s[pl.ds(eb + 16 * t, 16)]
            per_core(lambda: pltpu.async_copy(x0_hbm.at[bcols], bxg, gsem),
                     lambda: pltpu.async_copy(x1_hbm.at[bcols], bxg, gsem))
            if not first:
                pltpu.async_copy(np_hbm.at[bcols], bng, nsem)

        def wait_and_scale(B):
            bxg, brows, bcols, bvch, bnv, bng, gsem, nsem = B
            pltpu.make_async_copy(x0_hbm.at[bcols], bxg, gsem).wait()
            if not first:
                pltpu.make_async_copy(np_hbm.at[bcols], bng, nsem).wait()
                for t in range(C // 16):
                    sl = pl.ds(16 * t, 16)
                    bnv[sl] = bvch[sl] * bng[sl]
            for t in range(C // 16):
                v16g = bvch[pl.ds(16 * t, 16)]

                def scale(e, _, v16g=v16g, t=t):
                    vb = _lane_bcast(v16g, e)
                    row = 16 * t + e
                    for dd in range(DH // 16):
                        sl = pl.ds(16 * dd, 16)
                        bxg[row, sl] = bxg[row, sl] * vb
                    return 0
                lax.fori_loop(0, 16, scale, 0)

        def scatter_async(B):
            bxg, brows, bcols, bvch, bnv, bng, gsem, nsem = B
            hs = [pltpu.async_copy(bxg, acc_sh.at[brows], ssem, add=True),
                  pltpu.async_copy(bvch, ord_sh.at[brows], ssem, add=True)]
            if not first:
                hs.append(pltpu.async_copy(bnv, nacc_sh.at[brows], ssem,
                                           add=True))
            return hs

        def scatter_sync(B):
            bxg, brows, bcols, bvch, bnv, bng, gsem, nsem = B
            pltpu.sync_copy(bxg, acc_sh.at[brows], add=True)
            pltpu.sync_copy(bvch, ord_sh.at[brows], add=True)
            if not first:
                pltpu.sync_copy(bnv, nacc_sh.at[brows], add=True)

        def wait_scatter(B):
            bxg, brows, bcols, bvch, bnv, bng, gsem, nsem = B
            pltpu.make_async_copy(bxg, acc_sh.at[brows], ssem).wait()
            pltpu.make_async_copy(bvch, ord_sh.at[brows], ssem).wait()
            if not first:
                pltpu.make_async_copy(bnv, nacc_sh.at[brows], ssem).wait()

        def pairchunk(j2, _):
            ea = (2 * j2) * C
            prep_and_fire(ea, bufs[0])

            @pl.when(j2 > 0)
            def _():
                wait_scatter(bufs[1])
            prep_and_fire(ea + C, bufs[1])
            wait_and_scale(bufs[0])
            hs = scatter_async(bufs[0])
            wait_and_scale(bufs[1])
            for h in hs:
                h.wait()
            scatter_async(bufs[1])
            return 0
        lax.fori_loop(0, NCH // 2, pairchunk, 0)
        wait_scatter(bufs[1])
        prep_and_fire((NCH - 1) * C, bufs[0])
        wait_and_scale(bufs[0])
        scatter_sync(bufs[0])
        plsc.subcore_barrier()

        if not first:
            gb = _lane_bcast(params_v[pl.ds(0, 16)], 0)

        def rowchunk(k, _):
            r0 = rbase + k * C
            pltpu.sync_copy(acc_sh.at[pl.ds(r0, C), :], xg)
            per_core(
                lambda: pltpu.sync_copy(x0_hbm.at[pl.ds(r0, C), :], xg_b),
                lambda: pltpu.sync_copy(x1_hbm.at[pl.ds(r0, C), :], xg_b))
            if not first:
                pltpu.sync_copy(ordp_hbm.at[pl.ds(r0, C)], ordch)

            if first:
                def comb(r, _):
                    for dd in range(DH // 16):
                        sl = pl.ds(16 * dd, 16)
                        xg[r, sl] = xg[r, sl] - xg_b[r, sl]
                    return 0
                lax.fori_loop(0, C, comb, 0)
            else:
                for t in range(C // 16):
                    o16 = ordch[pl.ds(16 * t, 16)] + 1.0

                    def comb(r, _, t=t, o16=o16):
                        coef = _lane_bcast(o16, r)
                        row = 16 * t + r
                        for dd in range(DH // 16):
                            sl = pl.ds(16 * dd, 16)
                            xg[row, sl] = gb * (xg[row, sl] -
                                                coef * xg_b[row, sl])
                        return 0
                    lax.fori_loop(0, 16, comb, 0)
            per_core(
                lambda: pltpu.sync_copy(xg, e0_out.at[pl.ds(r0, C), :]),
                lambda: pltpu.sync_copy(xg, e1_out.at[pl.ds(r0, C), :]))

            @pl.when(cid == 0)
            def _():
                pltpu.sync_copy(ord_sh.at[pl.ds(r0, C)], vch)
                pltpu.sync_copy(vch, ord_out.at[pl.ds(r0, C)])
                if not first:
                    pltpu.sync_copy(nacc_sh.at[pl.ds(r0, C)], nvch)
                    pltpu.sync_copy(np_hbm.at[pl.ds(r0, C)], npch)
                    for t in range(C // 16):
                        sl = pl.ds(16 * t, 16)
                        nvch[sl] = gb * (nvch[sl] - npch[sl] - ordch[sl])
                    pltpu.sync_copy(nvch, n_out_hbm.at[pl.ds(r0, C)])
            return 0
        lax.fori_loop(0, RCH, rowchunk, 0)

    @functools.partial(pl.kernel, mesh=mesh, out_type=out_type,
                       scratch_types=scratch)
    def pass_kernel(rc_hbm, vals_hbm, ordp_hbm, np_hbm,
                    params_hbm, x0_hbm, x1_hbm, e0_out, e1_out, ord_out,
                    *rest):
        if first:
            n_out_hbm, scratch_refs = None, rest
        else:
            n_out_hbm, scratch_refs = rest[0], rest[1:]
        (rc_s, vals_s, xg, xg_b, rows_ch, cols_ch, vch, nvch,
         rows_chb, cols_chb, vch_b, nvch_b, ng_a, ng_b, ordch, npch,
         z1d, params_v, acc_sh, ord_sh, nacc_sh,
         gsem_a, gsem_b, nsem_a, nsem_b, ssem) = scratch_refs
        cid = lax.axis_index("c")
        sid = lax.axis_index("s")
        body(cid, sid, rc_hbm, vals_hbm, ordp_hbm, np_hbm, params_hbm,
             x0_hbm, x1_hbm, e0_out, e1_out, ord_out, n_out_hbm,
             rc_s, vals_s, xg, xg_b, rows_ch, cols_ch, vch, nvch,
             rows_chb, cols_chb, vch_b, nvch_b, ng_a, ng_b, ordch, npch,
             z1d, params_v, acc_sh, ord_sh, nacc_sh,
             gsem_a, gsem_b, nsem_a, nsem_b, ssem)

    return pass_kernel


_pass_first = _make_pass(True)
_pass_rest = _make_pass(False)


_TB = 256
_GRID = NPAD // _TB
_K = 100


def _score_body(e0l, e0h, e1l, e1h, e2l, e2h, e3l, e3h, emb, n0, n1, n2, n3,
                u, nmc, scores_out, cand_out, sacc):
    step = pl.program_id(0)
    lo = e0l[...] + e1l[...] + e2l[...] + e3l[...]
    hi = e0h[...] + e1h[...] + e2h[...] + e3h[...]
    nsum = n0[...] + n1[...] + n2[...] + n3[...] + 1e-8
    lo = lo / nsum
    hi = hi / nsum
    el = emb[:, :DH]
    eh = emb[:, DH:]
    dot = jnp.sum(lo * el + hi * eh, axis=1)
    ns = jnp.sqrt(jnp.sum(lo * lo + hi * hi, axis=1))
    ne = jnp.sqrt(jnp.sum(emb[...] * emb[...], axis=1))
    g = -jnp.log(-jnp.log(u[...][:, 0]))
    row = step * _TB + lax.broadcasted_iota(jnp.int32, (_TB,), 0)
    s = dot / (jnp.maximum(ns, 1e-12) * jnp.maximum(ne, 1e-12)) + g
    s = jnp.where(row < N, s, -1e30)
    scores_out[...] = s[:, None]
    sacc[step] = s

    @pl.when(step == _GRID - 1)
    def _():
        flat_iota = (lax.broadcasted_iota(jnp.int32, (_GRID, _TB), 0) * _TB +
                     lax.broadcasted_iota(jnp.int32, (_GRID, _TB), 1))
        lane = lax.broadcasted_iota(jnp.int32, (8, 128), 1)
        rowz = lax.broadcasted_iota(jnp.int32, (8, 128), 0)

        def pick(k, carry):
            sv, cand = carry
            m = jnp.max(sv)
            idx = jnp.min(jnp.where(sv == m, flat_iota, jnp.int32(2 ** 30)))
            cand = jnp.where((rowz == 0) & (lane == k), idx, cand)
            sv = jnp.where(flat_iota == idx, -3e30, sv)
            return sv, cand

        _, cand = lax.fori_loop(0, _K, pick,
                                (sacc[...], jnp.full((8, 128), -1,
                                                     jnp.int32)))
        cand = jnp.where(lane < nmc[0, 0], cand, -1)
        cand_out[...] = cand[0:1, :]


def _score_call(parts, embp, nvecs, u, nmc):
    row_spec = pl.BlockSpec((_TB, DH), lambda i: (i, 0))
    full_spec = pl.BlockSpec((_TB, D), lambda i: (i, 0))
    col_spec = pl.BlockSpec((_TB, 1), lambda i: (i, 0))
    one_spec = pl.BlockSpec((1, 1), lambda i: (0, 0))
    return pl.pallas_call(
        _score_body,
        grid=(_GRID,),
        in_specs=[row_spec] * 8 + [full_spec] + [col_spec] * 5 + [one_spec],
        out_specs=[pl.BlockSpec((_TB, 1), lambda i: (i, 0)),
                   pl.BlockSpec((1, 128), lambda i: (0, 0))],
        out_shape=[jax.ShapeDtypeStruct((NPAD, 1), jnp.float32),
                   jax.ShapeDtypeStruct((1, 128), jnp.int32)],
        scratch_shapes=[pltpu.VMEM((_GRID, _TB), jnp.float32)],
    )(*parts, embp, *nvecs, u, nmc)


def kernel(edge_index, adj_values, embeds, mask_depth, num_mask_cand):
    rows = edge_index[0].astype(jnp.int32)
    cols = edge_index[1].astype(jnp.int32)
    adj = adj_values.astype(jnp.float32)

    embp = jnp.pad(embeds, ((0, NPAD - N), (0, 0)))
    x0 = embp[:, :DH]
    x1 = embp[:, DH:]

    fs = []
    for i in range(3):
        keep = 0.5 ** (i + 1)
        un = jax.random.uniform(jax.random.fold_in(jax.random.key(1), i),
                                (E,))
        active = i < mask_depth
        f = jnp.where(active, jnp.where(un < keep, 1.0 / keep, 0.0), 1.0)
        fs.append(f)
    fc = [fs[0], fs[0] * fs[1], fs[0] * fs[1] * fs[2]]
    gflags = [jnp.where(i < mask_depth, 1.0, 0.0) for i in range(3)]
    params = [jnp.zeros((16,), jnp.float32).at[0].set(g) for g in gflags]
    unoise = jax.random.uniform(jax.random.key(2), (N,))
    up = jnp.pad(unoise, (0, NPAD - N), constant_values=0.5)[:, None]

    dummy_f = jnp.zeros((16,), jnp.float32)
    dummy_n = jnp.zeros((NPAD,), jnp.float32)
    rc = rows * 16384 + cols

    e0l, e0h, ord0 = _pass_first(rc, adj, dummy_n, dummy_n,
                                 dummy_f, x0, x1)
    n0 = ord0
    e1l, e1h, ord1, n1 = _pass_rest(rc, adj * fc[0], ord0, n0,
                                    params[0], e0l, e0h)
    e2l, e2h, ord2, n2 = _pass_rest(rc, adj * fc[1], ord1, n1,
                                    params[1], e1l, e1h)
    e3l, e3h, ord3, n3 = _pass_rest(rc, adj * fc[2], ord2, n2,
                                    params[2], e2l, e2h)

    nmc = jnp.full((1, 1), num_mask_cand, jnp.int32)
    scores_p, cand = _score_call(
        [e0l, e0h, e1l, e1h, e2l, e2h, e3l, e3h], embp,
        [n0[:, None], n1[:, None], n2[:, None], n3[:, None]], up, nmc)
    return scores_p[:N, 0], cand[0, :_K]
